# baseline (speedup 1.0000x reference)
import functools

import jax
import jax.numpy as jnp
from jax import lax
from jax.experimental import pallas as pl
from jax.experimental.pallas import tpu as pltpu

N_TOK = 4096
D_IN = 1024
E = 8
H = 1024
D_OUT = 1024
K = 2
LOSS_COEF = 0.01

BN_GATE = 1024
BM = 256
NB = 39
P = NB * BM
PB1 = 20
BN_MLP = 1024

_BF = jnp.bfloat16
_F32 = jnp.float32


def _gating_body(x_ref, wg_ref, i01_ref, g01_ref, rank01_ref, imp_ref,
                 load_ref, cnt_ref):
    t = pl.program_id(0)
    logits = jnp.dot(x_ref[...].astype(_BF), wg_ref[...].astype(_BF),
                     preferred_element_type=_F32)
    bn = logits.shape[0]
    iot = lax.broadcasted_iota(jnp.int32, (bn, E), 1)
    v0 = jnp.max(logits, axis=-1, keepdims=True)
    i0 = jnp.argmax(logits, axis=-1).astype(jnp.int32)
    masked = jnp.where(iot == i0[:, None], -jnp.inf, logits)
    v1 = jnp.max(masked, axis=-1, keepdims=True)
    i1 = jnp.argmax(masked, axis=-1).astype(jnp.int32)
    e1 = jnp.exp(v1 - v0)
    denom = 1.0 + e1
    g0 = 1.0 / denom
    g1 = e1 / denom
    i01_ref[...] = jnp.concatenate([i0[:, None], i1[:, None]], axis=1)
    g01_ref[...] = jnp.concatenate([g0, g1], axis=1)
    oh0 = (iot == i0[:, None]).astype(_F32)
    oh1 = (iot == i1[:, None]).astype(_F32)
    imp_blk = jnp.sum(oh0 * g0 + oh1 * g1, axis=0, keepdims=True)
    load_blk = jnp.sum(oh0 + oh1 * (g1 > 0.0).astype(_F32), axis=0,
                       keepdims=True)

    @pl.when(t == 0)
    def _():
        imp_ref[...] = jnp.zeros_like(imp_ref)
        load_ref[...] = jnp.zeros_like(load_ref)
        cnt_ref[...] = jnp.zeros_like(cnt_ref)

    base = cnt_ref[...]
    ohb = oh0 + oh1
    row = lax.broadcasted_iota(jnp.int32, (bn, bn), 0)
    col = lax.broadcasted_iota(jnp.int32, (bn, bn), 1)
    ltri = (row > col).astype(_F32)
    cex = jnp.dot(ltri, ohb, preferred_element_type=_F32)
    rank0 = jnp.sum((base + cex) * oh0, axis=1, keepdims=True)
    rank1 = jnp.sum((base + cex) * oh1, axis=1, keepdims=True)
    rank01_ref[...] = jnp.concatenate([rank0, rank1],
                                      axis=1).astype(jnp.int32)

    imp_ref[...] += imp_blk
    load_ref[...] += load_blk
    cnt_ref[...] += jnp.sum(ohb, axis=0, keepdims=True)


def _gating(x, w_gate):
    n = x.shape[0]
    grid = (n // BN_GATE,)
    return pl.pallas_call(
        _gating_body,
        grid=grid,
        in_specs=[
            pl.BlockSpec((BN_GATE, D_IN), lambda t: (t, 0)),
            pl.BlockSpec((D_IN, E), lambda t: (0, 0)),
        ],
        out_specs=[
            pl.BlockSpec((BN_GATE, K), lambda t: (t, 0)),
            pl.BlockSpec((BN_GATE, K), lambda t: (t, 0)),
            pl.BlockSpec((BN_GATE, K), lambda t: (t, 0)),
            pl.BlockSpec((1, E), lambda t: (0, 0)),
            pl.BlockSpec((1, E), lambda t: (0, 0)),
            pl.BlockSpec((1, E), lambda t: (0, 0)),
        ],
        out_shape=[
            jax.ShapeDtypeStruct((n, K), jnp.int32),
            jax.ShapeDtypeStruct((n, K), _F32),
            jax.ShapeDtypeStruct((n, K), jnp.int32),
            jax.ShapeDtypeStruct((1, E), _F32),
            jax.ShapeDtypeStruct((1, E), _F32),
            jax.ShapeDtypeStruct((1, E), _F32),
        ],
        compiler_params=pltpu.CompilerParams(
            dimension_semantics=("arbitrary",)),
    )(x, w_gate)


def _gmm_body(be_ref, xs_ref, w1_ref, b1_ref, w2_ref, b2_ref, eo_ref):
    h = jnp.dot(xs_ref[...], w1_ref[0], preferred_element_type=_F32)
    h = jnp.tanh(h + b1_ref[0])
    eo = jnp.dot(h.astype(_BF), w2_ref[0], preferred_element_type=_F32)
    eo_ref[...] = (eo + b2_ref[0]).astype(_BF)


def _gmm_chunk(xs, ew1, eb1, ew2, eb2, block_expert, nb, base, eo_prev):
    grid_spec = pltpu.PrefetchScalarGridSpec(
        num_scalar_prefetch=1,
        grid=(nb,),
        in_specs=[
            pl.BlockSpec((BM, D_IN), lambda i, be: (i, 0)),
            pl.BlockSpec((1, D_IN, H), lambda i, be: (be[i], 0, 0)),
            pl.BlockSpec((1, 1, H), lambda i, be: (be[i], 0, 0)),
            pl.BlockSpec((1, H, H), lambda i, be: (be[i], 0, 0)),
            pl.BlockSpec((1, 1, H), lambda i, be: (be[i], 0, 0)),
        ] + ([pl.BlockSpec(memory_space=pl.ANY)]
             if eo_prev is not None else []),
        out_specs=pl.BlockSpec((BM, H), lambda i, be: (base + i, 0)),
    )
    args = [block_expert, xs, ew1, eb1, ew2, eb2]
    kwargs = {}
    if eo_prev is not None:
        args.append(eo_prev)
        kwargs["input_output_aliases"] = {6: 0}

    def body(be_ref, xs_ref, w1_ref, b1_ref, w2_ref, b2_ref, *rest):
        eo_ref = rest[-1]
        _gmm_body(be_ref, xs_ref, w1_ref, b1_ref, w2_ref, b2_ref, eo_ref)

    return pl.pallas_call(
        body,
        grid_spec=grid_spec,
        out_shape=jax.ShapeDtypeStruct((P, H), _BF),
        compiler_params=pltpu.CompilerParams(
            dimension_semantics=("arbitrary",)),
        **kwargs,
    )(*args)


def _mlp_body(a_ref, b_ref, g_ref, mw1_ref, mb1_ref, mw2_ref, mb2_ref,
              fw_ref, fb_ref, out_ref):
    g = g_ref[...].astype(_BF).astype(_F32)
    g0 = g[:, 0:1]
    g1 = g[:, 1:2]
    y = a_ref[...].astype(_F32) * g0 + b_ref[...].astype(_F32) * g1
    y1 = jnp.tanh(jnp.dot(y.astype(_BF), mw1_ref[...],
                          preferred_element_type=_F32) + mb1_ref[...])
    y2 = jnp.tanh(jnp.dot(y1.astype(_BF), mw2_ref[...],
                          preferred_element_type=_F32) + mb2_ref[...])
    out_ref[...] = jnp.dot(y2.astype(_BF), fw_ref[...],
                           preferred_element_type=_F32) + fb_ref[...]


def _combine_mlp(a, b, g01, mw1, mb1, mw2, mb2, fw, fb, n_total, base,
                 out_prev):
    nc = a.shape[0]
    grid = (nc // BN_MLP,)

    def body(a_ref, b_ref, g_ref, mw1_ref, mb1_ref, mw2_ref, mb2_ref,
             fw_ref, fb_ref, *rest):
        _mlp_body(a_ref, b_ref, g_ref, mw1_ref, mb1_ref, mw2_ref, mb2_ref,
                  fw_ref, fb_ref, rest[-1])

    in_specs = [
        pl.BlockSpec((BN_MLP, H), lambda t: (t, 0)),
        pl.BlockSpec((BN_MLP, H), lambda t: (t, 0)),
        pl.BlockSpec((BN_MLP, K), lambda t: (t, 0)),
        pl.BlockSpec((H, H), lambda t: (0, 0)),
        pl.BlockSpec((1, H), lambda t: (0, 0)),
        pl.BlockSpec((H, H), lambda t: (0, 0)),
        pl.BlockSpec((1, H), lambda t: (0, 0)),
        pl.BlockSpec((H, D_OUT), lambda t: (0, 0)),
        pl.BlockSpec((1, D_OUT), lambda t: (0, 0)),
    ]
    args = [a, b, g01, mw1, mb1, mw2, mb2, fw, fb]
    kwargs = {}
    if out_prev is not None:
        in_specs.append(pl.BlockSpec(memory_space=pl.ANY))
        args.append(out_prev)
        kwargs["input_output_aliases"] = {9: 0}
    return pl.pallas_call(
        body,
        grid=grid,
        in_specs=in_specs,
        out_specs=pl.BlockSpec((BN_MLP, D_OUT), lambda t: (base + t, 0)),
        out_shape=jax.ShapeDtypeStruct((n_total, D_OUT), _F32),
        compiler_params=pltpu.CompilerParams(
            dimension_semantics=("arbitrary",)),
        **kwargs,
    )(*args)


def _cv_squared(v):
    eps = 1e-10
    return jnp.var(v, ddof=1) / (jnp.mean(v) ** 2 + eps)


def kernel(x, w_gate, ew1, eb1, ew2, eb2, mw1, mb1, mw2, mb2, fw, fb):
    n = x.shape[0]
    i01, g01, rank01, imp, load, cnt = _gating(x, w_gate)

    flat_e = i01.reshape(-1)
    counts = cnt.reshape(E).astype(jnp.int32)
    pc = ((counts + BM - 1) // BM) * BM
    poff = jnp.concatenate([jnp.zeros((1,), jnp.int32),
                            jnp.cumsum(pc)[:-1].astype(jnp.int32)])
    dest = poff[flat_e] + rank01.reshape(-1)
    pair_tok = jnp.arange(n * K, dtype=jnp.int32) // K
    src_row = jnp.zeros((P,), jnp.int32).at[dest].set(pair_tok)
    blk_start = jnp.arange(NB, dtype=jnp.int32) * BM
    block_expert = (jnp.sum(
        (blk_start[:, None] >= poff[None, :]).astype(jnp.int32), axis=1) - 1
    ).astype(jnp.int32)

    xb = x.astype(_BF)
    p1 = PB1 * BM
    xs_a = jnp.take(xb, src_row[:p1], axis=0)
    xs_b = jnp.take(xb, src_row[p1:], axis=0)

    ew1b, ew2b = ew1.astype(_BF), ew2.astype(_BF)
    eb1r, eb2r = eb1.reshape(E, 1, H), eb2.reshape(E, 1, H)
    eo_a = _gmm_chunk(xs_a, ew1b, eb1r, ew2b, eb2r, block_expert[:PB1],
                      PB1, 0, None)
    eo = _gmm_chunk(xs_b, ew1b, eb1r, ew2b, eb2r, block_expert[PB1:],
                    NB - PB1, PB1, eo_a)

    d2 = dest.reshape(n, K)
    n2 = n // 2
    idx1 = jnp.concatenate([d2[:n2, 0], d2[:n2, 1]])
    idx2 = jnp.concatenate([d2[n2:, 0], d2[n2:, 1]])
    comb1 = jnp.take(eo, idx1, axis=0)
    comb2 = jnp.take(eo, idx2, axis=0)

    mw1b, mw2b, fwb = mw1.astype(_BF), mw2.astype(_BF), fw.astype(_BF)
    mb1r, mb2r, fbr = mb1.reshape(1, H), mb2.reshape(1, H), fb.reshape(1,
                                                                       D_OUT)
    out1 = _combine_mlp(comb1[:n2], comb1[n2:], g01[:n2], mw1b, mb1r,
                        mw2b, mb2r, fwb, fbr, n, 0, None)
    out = _combine_mlp(comb2[:n2], comb2[n2:], g01[n2:], mw1b, mb1r,
                       mw2b, mb2r, fwb, fbr, n, n2 // BN_MLP, out1)

    aux = LOSS_COEF * (_cv_squared(imp.reshape(E)) +
                       _cv_squared(load.reshape(E)))
    return (out, aux)


# speedup vs baseline: 1.1029x; 1.1029x over previous
import functools

import jax
import jax.numpy as jnp
from jax import lax
from jax.experimental import pallas as pl
from jax.experimental.pallas import tpu as pltpu

N_TOK = 4096
D_IN = 1024
E = 8
H = 1024
D_OUT = 1024
K = 2
LOSS_COEF = 0.01

BN_GATE = 1024
BM = 256
NB = 39
P = NB * BM
PB1 = 20
BN_MLP = 1024

_BF = jnp.bfloat16
_F32 = jnp.float32


def _gating_body(x_ref, wg_ref, i01_ref, g01_ref, rank01_ref, imp_ref,
                 load_ref, cnt_ref):
    t = pl.program_id(0)
    logits = jnp.dot(x_ref[...].astype(_BF), wg_ref[...].astype(_BF),
                     preferred_element_type=_F32)
    bn = logits.shape[0]
    iot = lax.broadcasted_iota(jnp.int32, (bn, E), 1)
    v0 = jnp.max(logits, axis=-1, keepdims=True)
    i0 = jnp.argmax(logits, axis=-1).astype(jnp.int32)
    masked = jnp.where(iot == i0[:, None], -jnp.inf, logits)
    v1 = jnp.max(masked, axis=-1, keepdims=True)
    i1 = jnp.argmax(masked, axis=-1).astype(jnp.int32)
    e1 = jnp.exp(v1 - v0)
    denom = 1.0 + e1
    g0 = 1.0 / denom
    g1 = e1 / denom
    i01_ref[...] = jnp.concatenate([i0[:, None], i1[:, None]], axis=1)
    g01_ref[...] = jnp.concatenate([g0, g1], axis=1)
    oh0 = (iot == i0[:, None]).astype(_F32)
    oh1 = (iot == i1[:, None]).astype(_F32)
    imp_blk = jnp.sum(oh0 * g0 + oh1 * g1, axis=0, keepdims=True)
    load_blk = jnp.sum(oh0 + oh1 * (g1 > 0.0).astype(_F32), axis=0,
                       keepdims=True)

    @pl.when(t == 0)
    def _():
        imp_ref[...] = jnp.zeros_like(imp_ref)
        load_ref[...] = jnp.zeros_like(load_ref)
        cnt_ref[...] = jnp.zeros_like(cnt_ref)

    base = cnt_ref[...]
    ohb = oh0 + oh1
    row = lax.broadcasted_iota(jnp.int32, (bn, bn), 0)
    col = lax.broadcasted_iota(jnp.int32, (bn, bn), 1)
    ltri = (row > col).astype(_F32)
    cex = jnp.dot(ltri, ohb, preferred_element_type=_F32)
    rank0 = jnp.sum((base + cex) * oh0, axis=1, keepdims=True)
    rank1 = jnp.sum((base + cex) * oh1, axis=1, keepdims=True)
    rank01_ref[...] = jnp.concatenate([rank0, rank1],
                                      axis=1).astype(jnp.int32)

    imp_ref[...] += imp_blk
    load_ref[...] += load_blk
    cnt_ref[...] += jnp.sum(ohb, axis=0, keepdims=True)


def _gating(x, w_gate):
    n = x.shape[0]
    grid = (n // BN_GATE,)
    return pl.pallas_call(
        _gating_body,
        grid=grid,
        in_specs=[
            pl.BlockSpec((BN_GATE, D_IN), lambda t: (t, 0)),
            pl.BlockSpec((D_IN, E), lambda t: (0, 0)),
        ],
        out_specs=[
            pl.BlockSpec((BN_GATE, K), lambda t: (t, 0)),
            pl.BlockSpec((BN_GATE, K), lambda t: (t, 0)),
            pl.BlockSpec((BN_GATE, K), lambda t: (t, 0)),
            pl.BlockSpec((1, E), lambda t: (0, 0)),
            pl.BlockSpec((1, E), lambda t: (0, 0)),
            pl.BlockSpec((1, E), lambda t: (0, 0)),
        ],
        out_shape=[
            jax.ShapeDtypeStruct((n, K), jnp.int32),
            jax.ShapeDtypeStruct((n, K), _F32),
            jax.ShapeDtypeStruct((n, K), jnp.int32),
            jax.ShapeDtypeStruct((1, E), _F32),
            jax.ShapeDtypeStruct((1, E), _F32),
            jax.ShapeDtypeStruct((1, E), _F32),
        ],
        compiler_params=pltpu.CompilerParams(
            dimension_semantics=("arbitrary",)),
    )(x, w_gate)


def _gmm_body(be_ref, xs_ref, w1_ref, b1_ref, w2_ref, b2_ref, eo_ref):
    h = jnp.dot(xs_ref[...], w1_ref[0], preferred_element_type=_F32)
    h = jnp.tanh(h + b1_ref[0])
    eo = jnp.dot(h.astype(_BF), w2_ref[0], preferred_element_type=_F32)
    eo_ref[...] = (eo + b2_ref[0]).astype(_BF)


def _gmm_chunk(xs, ew1, eb1, ew2, eb2, block_expert, nb, base, eo_prev):
    grid_spec = pltpu.PrefetchScalarGridSpec(
        num_scalar_prefetch=1,
        grid=(nb,),
        in_specs=[
            pl.BlockSpec((BM, D_IN), lambda i, be: (i, 0)),
            pl.BlockSpec((1, D_IN, H), lambda i, be: (be[i], 0, 0)),
            pl.BlockSpec((1, 1, H), lambda i, be: (be[i], 0, 0)),
            pl.BlockSpec((1, H, H), lambda i, be: (be[i], 0, 0)),
            pl.BlockSpec((1, 1, H), lambda i, be: (be[i], 0, 0)),
        ] + ([pl.BlockSpec(memory_space=pl.ANY)]
             if eo_prev is not None else []),
        out_specs=pl.BlockSpec((BM, H), lambda i, be: (base + i, 0)),
    )
    args = [block_expert, xs, ew1, eb1, ew2, eb2]
    kwargs = {}
    if eo_prev is not None:
        args.append(eo_prev)
        kwargs["input_output_aliases"] = {6: 0}

    def body(be_ref, xs_ref, w1_ref, b1_ref, w2_ref, b2_ref, *rest):
        eo_ref = rest[-1]
        _gmm_body(be_ref, xs_ref, w1_ref, b1_ref, w2_ref, b2_ref, eo_ref)

    return pl.pallas_call(
        body,
        grid_spec=grid_spec,
        out_shape=jax.ShapeDtypeStruct((P, H), _BF),
        compiler_params=pltpu.CompilerParams(
            dimension_semantics=("arbitrary",)),
        **kwargs,
    )(*args)


def _mlp_body(a_ref, b_ref, g_ref, mw1_ref, mb1_ref, mw2_ref, mb2_ref,
              fw_ref, fb_ref, out_ref):
    g = g_ref[...].astype(_BF).astype(_F32)
    g0 = g[:, 0:1]
    g1 = g[:, 1:2]
    y = a_ref[...].astype(_F32) * g0 + b_ref[...].astype(_F32) * g1
    y1 = jnp.tanh(jnp.dot(y.astype(_BF), mw1_ref[...],
                          preferred_element_type=_F32) + mb1_ref[...])
    y2 = jnp.tanh(jnp.dot(y1.astype(_BF), mw2_ref[...],
                          preferred_element_type=_F32) + mb2_ref[...])
    out_ref[...] = jnp.dot(y2.astype(_BF), fw_ref[...],
                           preferred_element_type=_F32) + fb_ref[...]


def _combine_mlp(a, b, g01, mw1, mb1, mw2, mb2, fw, fb, n_total, base,
                 out_prev):
    nc = a.shape[0]
    grid = (nc // BN_MLP,)

    def body(a_ref, b_ref, g_ref, mw1_ref, mb1_ref, mw2_ref, mb2_ref,
             fw_ref, fb_ref, *rest):
        _mlp_body(a_ref, b_ref, g_ref, mw1_ref, mb1_ref, mw2_ref, mb2_ref,
                  fw_ref, fb_ref, rest[-1])

    in_specs = [
        pl.BlockSpec((BN_MLP, H), lambda t: (t, 0)),
        pl.BlockSpec((BN_MLP, H), lambda t: (t, 0)),
        pl.BlockSpec((BN_MLP, K), lambda t: (t, 0)),
        pl.BlockSpec((H, H), lambda t: (0, 0)),
        pl.BlockSpec((1, H), lambda t: (0, 0)),
        pl.BlockSpec((H, H), lambda t: (0, 0)),
        pl.BlockSpec((1, H), lambda t: (0, 0)),
        pl.BlockSpec((H, D_OUT), lambda t: (0, 0)),
        pl.BlockSpec((1, D_OUT), lambda t: (0, 0)),
    ]
    args = [a, b, g01, mw1, mb1, mw2, mb2, fw, fb]
    kwargs = {}
    if out_prev is not None:
        in_specs.append(pl.BlockSpec(memory_space=pl.ANY))
        args.append(out_prev)
        kwargs["input_output_aliases"] = {9: 0}
    return pl.pallas_call(
        body,
        grid=grid,
        in_specs=in_specs,
        out_specs=pl.BlockSpec((BN_MLP, D_OUT), lambda t: (base + t, 0)),
        out_shape=jax.ShapeDtypeStruct((n_total, D_OUT), _F32),
        compiler_params=pltpu.CompilerParams(
            dimension_semantics=("arbitrary",)),
        **kwargs,
    )(*args)


def _cv_squared(v):
    eps = 1e-10
    return jnp.var(v, ddof=1) / (jnp.mean(v) ** 2 + eps)


def kernel(x, w_gate, ew1, eb1, ew2, eb2, mw1, mb1, mw2, mb2, fw, fb):
    n = x.shape[0]
    i01, g01, rank01, imp, load, cnt = _gating(x, w_gate)

    flat_e = i01.reshape(-1)
    counts = cnt.reshape(E).astype(jnp.int32)
    pc = ((counts + BM - 1) // BM) * BM
    poff = jnp.concatenate([jnp.zeros((1,), jnp.int32),
                            jnp.cumsum(pc)[:-1].astype(jnp.int32)])
    dest = poff[flat_e] + rank01.reshape(-1)
    pair_tok = jnp.arange(n * K, dtype=jnp.int32) // K
    src_row = jnp.zeros((P,), jnp.int32).at[dest].set(pair_tok)
    blk_start = jnp.arange(NB, dtype=jnp.int32) * BM
    block_expert = (jnp.sum(
        (blk_start[:, None] >= poff[None, :]).astype(jnp.int32), axis=1) - 1
    ).astype(jnp.int32)

    xb = x.astype(_BF)
    xs = jnp.take(xb, src_row, axis=0)

    eo = _gmm_chunk(xs, ew1.astype(_BF), eb1.reshape(E, 1, H),
                    ew2.astype(_BF), eb2.reshape(E, 1, H), block_expert,
                    NB, 0, None)

    d2 = dest.reshape(n, K)
    comb_idx = jnp.concatenate([d2[:, 0], d2[:, 1]])
    comb = jnp.take(eo, comb_idx, axis=0)

    out = _combine_mlp(comb[:n], comb[n:], g01, mw1.astype(_BF),
                       mb1.reshape(1, H), mw2.astype(_BF),
                       mb2.reshape(1, H), fw.astype(_BF),
                       fb.reshape(1, D_OUT), n, 0, None)

    aux = LOSS_COEF * (_cv_squared(imp.reshape(E)) +
                       _cv_squared(load.reshape(E)))
    return (out, aux)


# speedup vs baseline: 1.1543x; 1.0467x over previous
import functools

import jax
import jax.numpy as jnp
from jax import lax
from jax.experimental import pallas as pl
from jax.experimental.pallas import tpu as pltpu

N_TOK = 4096
D_IN = 1024
E = 8
H = 1024
D_OUT = 1024
K = 2
LOSS_COEF = 0.01

BN_GATE = 1024
BM = 256
NB = 39
P = NB * BM
PB1 = 20
BN_MLP = 1024

_BF = jnp.bfloat16
_F32 = jnp.float32


def _gating_body(x_ref, wg_ref, i01_ref, g01_ref, rank01_ref, imp_ref,
                 load_ref, cnt_ref):
    t = pl.program_id(0)
    logits = jnp.dot(x_ref[...].astype(_BF), wg_ref[...].astype(_BF),
                     preferred_element_type=_F32)
    bn = logits.shape[0]
    iot = lax.broadcasted_iota(jnp.int32, (bn, E), 1)
    v0 = jnp.max(logits, axis=-1, keepdims=True)
    i0 = jnp.argmax(logits, axis=-1).astype(jnp.int32)
    masked = jnp.where(iot == i0[:, None], -jnp.inf, logits)
    v1 = jnp.max(masked, axis=-1, keepdims=True)
    i1 = jnp.argmax(masked, axis=-1).astype(jnp.int32)
    e1 = jnp.exp(v1 - v0)
    denom = 1.0 + e1
    g0 = 1.0 / denom
    g1 = e1 / denom
    i01_ref[...] = jnp.concatenate([i0[:, None], i1[:, None]], axis=1)
    g01_ref[...] = jnp.concatenate([g0, g1], axis=1)
    oh0 = (iot == i0[:, None]).astype(_F32)
    oh1 = (iot == i1[:, None]).astype(_F32)
    imp_blk = jnp.sum(oh0 * g0 + oh1 * g1, axis=0, keepdims=True)
    load_blk = jnp.sum(oh0 + oh1 * (g1 > 0.0).astype(_F32), axis=0,
                       keepdims=True)

    @pl.when(t == 0)
    def _():
        imp_ref[...] = jnp.zeros_like(imp_ref)
        load_ref[...] = jnp.zeros_like(load_ref)
        cnt_ref[...] = jnp.zeros_like(cnt_ref)

    base = cnt_ref[...]
    ohb = oh0 + oh1
    row = lax.broadcasted_iota(jnp.int32, (bn, bn), 0)
    col = lax.broadcasted_iota(jnp.int32, (bn, bn), 1)
    ltri = (row > col).astype(_F32)
    cex = jnp.dot(ltri, ohb, preferred_element_type=_F32)
    rank0 = jnp.sum((base + cex) * oh0, axis=1, keepdims=True)
    rank1 = jnp.sum((base + cex) * oh1, axis=1, keepdims=True)
    rank01_ref[...] = jnp.concatenate([rank0, rank1],
                                      axis=1).astype(jnp.int32)

    imp_ref[...] += imp_blk
    load_ref[...] += load_blk
    cnt_ref[...] += jnp.sum(ohb, axis=0, keepdims=True)


def _gating(x, w_gate):
    n = x.shape[0]
    grid = (n // BN_GATE,)
    return pl.pallas_call(
        _gating_body,
        grid=grid,
        in_specs=[
            pl.BlockSpec((BN_GATE, D_IN), lambda t: (t, 0)),
            pl.BlockSpec((D_IN, E), lambda t: (0, 0)),
        ],
        out_specs=[
            pl.BlockSpec((BN_GATE, K), lambda t: (t, 0)),
            pl.BlockSpec((BN_GATE, K), lambda t: (t, 0)),
            pl.BlockSpec((BN_GATE, K), lambda t: (t, 0)),
            pl.BlockSpec((1, E), lambda t: (0, 0)),
            pl.BlockSpec((1, E), lambda t: (0, 0)),
            pl.BlockSpec((1, E), lambda t: (0, 0)),
        ],
        out_shape=[
            jax.ShapeDtypeStruct((n, K), jnp.int32),
            jax.ShapeDtypeStruct((n, K), _F32),
            jax.ShapeDtypeStruct((n, K), jnp.int32),
            jax.ShapeDtypeStruct((1, E), _F32),
            jax.ShapeDtypeStruct((1, E), _F32),
            jax.ShapeDtypeStruct((1, E), _F32),
        ],
        compiler_params=pltpu.CompilerParams(
            dimension_semantics=("arbitrary",)),
    )(x, w_gate)


def _gmm_body(be_ref, xs_ref, w1_ref, b1_ref, w2_ref, b2_ref, eo_ref,
              w1s_ref, w2s_ref):
    i = pl.program_id(0)
    prev = be_ref[jnp.maximum(i - 1, 0)]

    @pl.when((i == 0) | (be_ref[i] != prev))
    def _():
        w1s_ref[...] = w1_ref[0].astype(_BF)
        w2s_ref[...] = w2_ref[0].astype(_BF)

    h = jnp.dot(xs_ref[...], w1s_ref[...], preferred_element_type=_F32)
    h = jnp.tanh(h + b1_ref[0])
    eo = jnp.dot(h.astype(_BF), w2s_ref[...], preferred_element_type=_F32)
    eo_ref[...] = (eo + b2_ref[0]).astype(_BF)


def _gmm_chunk(xs, ew1, eb1, ew2, eb2, block_expert, nb, base, eo_prev):
    grid_spec = pltpu.PrefetchScalarGridSpec(
        num_scalar_prefetch=1,
        grid=(nb,),
        in_specs=[
            pl.BlockSpec((BM, D_IN), lambda i, be: (i, 0)),
            pl.BlockSpec((1, D_IN, H), lambda i, be: (be[i], 0, 0)),
            pl.BlockSpec((1, 1, H), lambda i, be: (be[i], 0, 0)),
            pl.BlockSpec((1, H, H), lambda i, be: (be[i], 0, 0)),
            pl.BlockSpec((1, 1, H), lambda i, be: (be[i], 0, 0)),
        ] + ([pl.BlockSpec(memory_space=pl.ANY)]
             if eo_prev is not None else []),
        out_specs=pl.BlockSpec((BM, H), lambda i, be: (base + i, 0)),
        scratch_shapes=[pltpu.VMEM((D_IN, H), _BF), pltpu.VMEM((H, H), _BF)],
    )
    args = [block_expert, xs, ew1, eb1, ew2, eb2]
    kwargs = {}
    if eo_prev is not None:
        args.append(eo_prev)
        kwargs["input_output_aliases"] = {6: 0}

    def body(be_ref, xs_ref, w1_ref, b1_ref, w2_ref, b2_ref, *rest):
        w1s_ref, w2s_ref = rest[-2], rest[-1]
        eo_ref = rest[-3]
        _gmm_body(be_ref, xs_ref, w1_ref, b1_ref, w2_ref, b2_ref, eo_ref,
                  w1s_ref, w2s_ref)

    return pl.pallas_call(
        body,
        grid_spec=grid_spec,
        out_shape=jax.ShapeDtypeStruct((P, H), _BF),
        compiler_params=pltpu.CompilerParams(
            dimension_semantics=("arbitrary",)),
        **kwargs,
    )(*args)


def _mlp_body(a_ref, b_ref, g_ref, mw1_ref, mb1_ref, mw2_ref, mb2_ref,
              fw_ref, fb_ref, out_ref, w1s_ref, w2s_ref, fws_ref):
    @pl.when(pl.program_id(0) == 0)
    def _():
        w1s_ref[...] = mw1_ref[...].astype(_BF)
        w2s_ref[...] = mw2_ref[...].astype(_BF)
        fws_ref[...] = fw_ref[...].astype(_BF)

    g = g_ref[...].astype(_BF).astype(_F32)
    g0 = g[:, 0:1]
    g1 = g[:, 1:2]
    y = a_ref[...].astype(_F32) * g0 + b_ref[...].astype(_F32) * g1
    y1 = jnp.tanh(jnp.dot(y.astype(_BF), w1s_ref[...],
                          preferred_element_type=_F32) + mb1_ref[...])
    y2 = jnp.tanh(jnp.dot(y1.astype(_BF), w2s_ref[...],
                          preferred_element_type=_F32) + mb2_ref[...])
    out_ref[...] = jnp.dot(y2.astype(_BF), fws_ref[...],
                           preferred_element_type=_F32) + fb_ref[...]


def _combine_mlp(a, b, g01, mw1, mb1, mw2, mb2, fw, fb, n_total, base,
                 out_prev):
    nc = a.shape[0]
    grid = (nc // BN_MLP,)

    def body(a_ref, b_ref, g_ref, mw1_ref, mb1_ref, mw2_ref, mb2_ref,
             fw_ref, fb_ref, *rest):
        _mlp_body(a_ref, b_ref, g_ref, mw1_ref, mb1_ref, mw2_ref, mb2_ref,
                  fw_ref, fb_ref, rest[-4], rest[-3], rest[-2], rest[-1])

    in_specs = [
        pl.BlockSpec((BN_MLP, H), lambda t: (t, 0)),
        pl.BlockSpec((BN_MLP, H), lambda t: (t, 0)),
        pl.BlockSpec((BN_MLP, K), lambda t: (t, 0)),
        pl.BlockSpec((H, H), lambda t: (0, 0)),
        pl.BlockSpec((1, H), lambda t: (0, 0)),
        pl.BlockSpec((H, H), lambda t: (0, 0)),
        pl.BlockSpec((1, H), lambda t: (0, 0)),
        pl.BlockSpec((H, D_OUT), lambda t: (0, 0)),
        pl.BlockSpec((1, D_OUT), lambda t: (0, 0)),
    ]
    args = [a, b, g01, mw1, mb1, mw2, mb2, fw, fb]
    kwargs = {}
    if out_prev is not None:
        in_specs.append(pl.BlockSpec(memory_space=pl.ANY))
        args.append(out_prev)
        kwargs["input_output_aliases"] = {9: 0}
    return pl.pallas_call(
        body,
        grid=grid,
        in_specs=in_specs,
        out_specs=pl.BlockSpec((BN_MLP, D_OUT), lambda t: (base + t, 0)),
        out_shape=jax.ShapeDtypeStruct((n_total, D_OUT), _F32),
        scratch_shapes=[pltpu.VMEM((H, H), _BF), pltpu.VMEM((H, H), _BF),
                        pltpu.VMEM((H, D_OUT), _BF)],
        compiler_params=pltpu.CompilerParams(
            dimension_semantics=("arbitrary",)),
        **kwargs,
    )(*args)


def _cv_squared(v):
    eps = 1e-10
    return jnp.var(v, ddof=1) / (jnp.mean(v) ** 2 + eps)


def kernel(x, w_gate, ew1, eb1, ew2, eb2, mw1, mb1, mw2, mb2, fw, fb):
    n = x.shape[0]
    i01, g01, rank01, imp, load, cnt = _gating(x, w_gate)

    flat_e = i01.reshape(-1)
    counts = cnt.reshape(E).astype(jnp.int32)
    pc = ((counts + BM - 1) // BM) * BM
    poff = jnp.concatenate([jnp.zeros((1,), jnp.int32),
                            jnp.cumsum(pc)[:-1].astype(jnp.int32)])
    dest = poff[flat_e] + rank01.reshape(-1)
    pair_tok = jnp.arange(n * K, dtype=jnp.int32) // K
    src_row = jnp.zeros((P,), jnp.int32).at[dest].set(pair_tok)
    blk_start = jnp.arange(NB, dtype=jnp.int32) * BM
    block_expert = (jnp.sum(
        (blk_start[:, None] >= poff[None, :]).astype(jnp.int32), axis=1) - 1
    ).astype(jnp.int32)

    xb = x.astype(_BF)
    xs = jnp.take(xb, src_row, axis=0)

    eo = _gmm_chunk(xs, ew1, eb1.reshape(E, 1, H),
                    ew2, eb2.reshape(E, 1, H), block_expert, NB, 0, None)

    d2 = dest.reshape(n, K)
    comb_idx = jnp.concatenate([d2[:, 0], d2[:, 1]])
    comb = jnp.take(eo, comb_idx, axis=0)

    out = _combine_mlp(comb[:n], comb[n:], g01, mw1, mb1.reshape(1, H),
                       mw2, mb2.reshape(1, H), fw,
                       fb.reshape(1, D_OUT), n, 0, None)

    aux = LOSS_COEF * (_cv_squared(imp.reshape(E)) +
                       _cv_squared(load.reshape(E)))
    return (out, aux)


# speedup vs baseline: 1.1888x; 1.0299x over previous
import functools

import jax
import jax.numpy as jnp
from jax import lax
from jax.experimental import pallas as pl
from jax.experimental.pallas import tpu as pltpu

N_TOK = 4096
D_IN = 1024
E = 8
H = 1024
D_OUT = 1024
K = 2
LOSS_COEF = 0.01

BN_GATE = 1024
BM = 256
NB = 39
P = NB * BM
PB1 = 20
BN_MLP = 1024

_BF = jnp.bfloat16
_F32 = jnp.float32


def _gating_body(x_ref, wg_ref, i01_ref, g01_ref, rank01_ref, imp_ref,
                 load_ref, cnt_ref, xb_ref):
    t = pl.program_id(0)
    xbf = x_ref[...].astype(_BF)
    xb_ref[...] = xbf
    logits = jnp.dot(xbf, wg_ref[...].astype(_BF),
                     preferred_element_type=_F32)
    bn = logits.shape[0]
    iot = lax.broadcasted_iota(jnp.int32, (bn, E), 1)
    v0 = jnp.max(logits, axis=-1, keepdims=True)
    i0 = jnp.argmax(logits, axis=-1).astype(jnp.int32)
    masked = jnp.where(iot == i0[:, None], -jnp.inf, logits)
    v1 = jnp.max(masked, axis=-1, keepdims=True)
    i1 = jnp.argmax(masked, axis=-1).astype(jnp.int32)
    e1 = jnp.exp(v1 - v0)
    denom = 1.0 + e1
    g0 = 1.0 / denom
    g1 = e1 / denom
    i01_ref[...] = jnp.concatenate([i0[:, None], i1[:, None]], axis=1)
    g01_ref[...] = jnp.concatenate([g0, g1], axis=1)
    oh0 = (iot == i0[:, None]).astype(_F32)
    oh1 = (iot == i1[:, None]).astype(_F32)
    imp_blk = jnp.sum(oh0 * g0 + oh1 * g1, axis=0, keepdims=True)
    load_blk = jnp.sum(oh0 + oh1 * (g1 > 0.0).astype(_F32), axis=0,
                       keepdims=True)

    @pl.when(t == 0)
    def _():
        imp_ref[...] = jnp.zeros_like(imp_ref)
        load_ref[...] = jnp.zeros_like(load_ref)
        cnt_ref[...] = jnp.zeros_like(cnt_ref)

    base = cnt_ref[...]
    ohb = oh0 + oh1
    row = lax.broadcasted_iota(jnp.int32, (bn, bn), 0)
    col = lax.broadcasted_iota(jnp.int32, (bn, bn), 1)
    ltri = (row > col).astype(_F32)
    cex = jnp.dot(ltri, ohb, preferred_element_type=_F32)
    rank0 = jnp.sum((base + cex) * oh0, axis=1, keepdims=True)
    rank1 = jnp.sum((base + cex) * oh1, axis=1, keepdims=True)
    rank01_ref[...] = jnp.concatenate([rank0, rank1],
                                      axis=1).astype(jnp.int32)

    imp_ref[...] += imp_blk
    load_ref[...] += load_blk
    cnt_ref[...] += jnp.sum(ohb, axis=0, keepdims=True)


def _gating(x, w_gate):
    n = x.shape[0]
    grid = (n // BN_GATE,)
    return pl.pallas_call(
        _gating_body,
        grid=grid,
        in_specs=[
            pl.BlockSpec((BN_GATE, D_IN), lambda t: (t, 0)),
            pl.BlockSpec((D_IN, E), lambda t: (0, 0)),
        ],
        out_specs=[
            pl.BlockSpec((BN_GATE, K), lambda t: (t, 0)),
            pl.BlockSpec((BN_GATE, K), lambda t: (t, 0)),
            pl.BlockSpec((BN_GATE, K), lambda t: (t, 0)),
            pl.BlockSpec((1, E), lambda t: (0, 0)),
            pl.BlockSpec((1, E), lambda t: (0, 0)),
            pl.BlockSpec((1, E), lambda t: (0, 0)),
            pl.BlockSpec((BN_GATE, D_IN), lambda t: (t, 0)),
        ],
        out_shape=[
            jax.ShapeDtypeStruct((n, K), jnp.int32),
            jax.ShapeDtypeStruct((n, K), _F32),
            jax.ShapeDtypeStruct((n, K), jnp.int32),
            jax.ShapeDtypeStruct((1, E), _F32),
            jax.ShapeDtypeStruct((1, E), _F32),
            jax.ShapeDtypeStruct((1, E), _F32),
            jax.ShapeDtypeStruct((n, D_IN), _BF),
        ],
        compiler_params=pltpu.CompilerParams(
            dimension_semantics=("arbitrary",)),
    )(x, w_gate)


def _gmm_body(be_ref, xs_ref, w1_ref, b1_ref, w2_ref, b2_ref, eo_ref,
              w1s_ref, w2s_ref):
    i = pl.program_id(0)
    prev = be_ref[jnp.maximum(i - 1, 0)]

    @pl.when((i == 0) | (be_ref[i] != prev))
    def _():
        w1s_ref[...] = w1_ref[0].astype(_BF)
        w2s_ref[...] = w2_ref[0].astype(_BF)

    h = jnp.dot(xs_ref[...], w1s_ref[...], preferred_element_type=_F32)
    h = jnp.tanh(h + b1_ref[0])
    eo = jnp.dot(h.astype(_BF), w2s_ref[...], preferred_element_type=_F32)
    eo_ref[...] = (eo + b2_ref[0]).astype(_BF)


def _gmm_chunk(xs, ew1, eb1, ew2, eb2, block_expert, nb, base, eo_prev):
    grid_spec = pltpu.PrefetchScalarGridSpec(
        num_scalar_prefetch=1,
        grid=(nb,),
        in_specs=[
            pl.BlockSpec((BM, D_IN), lambda i, be: (i, 0)),
            pl.BlockSpec((1, D_IN, H), lambda i, be: (be[i], 0, 0)),
            pl.BlockSpec((1, 1, H), lambda i, be: (be[i], 0, 0)),
            pl.BlockSpec((1, H, H), lambda i, be: (be[i], 0, 0)),
            pl.BlockSpec((1, 1, H), lambda i, be: (be[i], 0, 0)),
        ] + ([pl.BlockSpec(memory_space=pl.ANY)]
             if eo_prev is not None else []),
        out_specs=pl.BlockSpec((BM, H), lambda i, be: (base + i, 0)),
        scratch_shapes=[pltpu.VMEM((D_IN, H), _BF), pltpu.VMEM((H, H), _BF)],
    )
    args = [block_expert, xs, ew1, eb1, ew2, eb2]
    kwargs = {}
    if eo_prev is not None:
        args.append(eo_prev)
        kwargs["input_output_aliases"] = {6: 0}

    def body(be_ref, xs_ref, w1_ref, b1_ref, w2_ref, b2_ref, *rest):
        w1s_ref, w2s_ref = rest[-2], rest[-1]
        eo_ref = rest[-3]
        _gmm_body(be_ref, xs_ref, w1_ref, b1_ref, w2_ref, b2_ref, eo_ref,
                  w1s_ref, w2s_ref)

    return pl.pallas_call(
        body,
        grid_spec=grid_spec,
        out_shape=jax.ShapeDtypeStruct((P, H), _BF),
        compiler_params=pltpu.CompilerParams(
            dimension_semantics=("arbitrary",)),
        **kwargs,
    )(*args)


def _mlp_body(a_ref, b_ref, g_ref, mw1_ref, mb1_ref, mw2_ref, mb2_ref,
              fw_ref, fb_ref, out_ref, w1s_ref, w2s_ref, fws_ref):
    @pl.when(pl.program_id(0) == 0)
    def _():
        w1s_ref[...] = mw1_ref[...].astype(_BF)
        w2s_ref[...] = mw2_ref[...].astype(_BF)
        fws_ref[...] = fw_ref[...].astype(_BF)

    g = g_ref[...].astype(_BF).astype(_F32)
    g0 = g[:, 0:1]
    g1 = g[:, 1:2]
    y = a_ref[...].astype(_F32) * g0 + b_ref[...].astype(_F32) * g1
    y1 = jnp.tanh(jnp.dot(y.astype(_BF), w1s_ref[...],
                          preferred_element_type=_F32) + mb1_ref[...])
    y2 = jnp.tanh(jnp.dot(y1.astype(_BF), w2s_ref[...],
                          preferred_element_type=_F32) + mb2_ref[...])
    out_ref[...] = jnp.dot(y2.astype(_BF), fws_ref[...],
                           preferred_element_type=_F32) + fb_ref[...]


def _combine_mlp(comb, g01, mw1, mb1, mw2, mb2, fw, fb, n_total, base,
                 out_prev):
    nc = comb.shape[0] // 2
    nblk = nc // BN_MLP
    grid = (nblk,)

    def body(a_ref, b_ref, g_ref, mw1_ref, mb1_ref, mw2_ref, mb2_ref,
             fw_ref, fb_ref, *rest):
        _mlp_body(a_ref, b_ref, g_ref, mw1_ref, mb1_ref, mw2_ref, mb2_ref,
                  fw_ref, fb_ref, rest[-4], rest[-3], rest[-2], rest[-1])

    in_specs = [
        pl.BlockSpec((BN_MLP, H), lambda t: (t, 0)),
        pl.BlockSpec((BN_MLP, H), lambda t: (nblk + t, 0)),
        pl.BlockSpec((BN_MLP, K), lambda t: (t, 0)),
        pl.BlockSpec((H, H), lambda t: (0, 0)),
        pl.BlockSpec((1, H), lambda t: (0, 0)),
        pl.BlockSpec((H, H), lambda t: (0, 0)),
        pl.BlockSpec((1, H), lambda t: (0, 0)),
        pl.BlockSpec((H, D_OUT), lambda t: (0, 0)),
        pl.BlockSpec((1, D_OUT), lambda t: (0, 0)),
    ]
    args = [comb, comb, g01, mw1, mb1, mw2, mb2, fw, fb]
    kwargs = {}
    if out_prev is not None:
        in_specs.append(pl.BlockSpec(memory_space=pl.ANY))
        args.append(out_prev)
        kwargs["input_output_aliases"] = {9: 0}
    return pl.pallas_call(
        body,
        grid=grid,
        in_specs=in_specs,
        out_specs=pl.BlockSpec((BN_MLP, D_OUT), lambda t: (base + t, 0)),
        out_shape=jax.ShapeDtypeStruct((n_total, D_OUT), _F32),
        scratch_shapes=[pltpu.VMEM((H, H), _BF), pltpu.VMEM((H, H), _BF),
                        pltpu.VMEM((H, D_OUT), _BF)],
        compiler_params=pltpu.CompilerParams(
            dimension_semantics=("arbitrary",)),
        **kwargs,
    )(*args)


def _cv_squared(v):
    eps = 1e-10
    return jnp.var(v, ddof=1) / (jnp.mean(v) ** 2 + eps)


def kernel(x, w_gate, ew1, eb1, ew2, eb2, mw1, mb1, mw2, mb2, fw, fb):
    n = x.shape[0]
    i01, g01, rank01, imp, load, cnt, xb = _gating(x, w_gate)

    flat_e = i01.reshape(-1)
    counts = cnt.reshape(E).astype(jnp.int32)
    pc = ((counts + BM - 1) // BM) * BM
    poff = jnp.concatenate([jnp.zeros((1,), jnp.int32),
                            jnp.cumsum(pc)[:-1].astype(jnp.int32)])
    dest = poff[flat_e] + rank01.reshape(-1)
    pair_tok = jnp.arange(n * K, dtype=jnp.int32) // K
    src_row = jnp.zeros((P,), jnp.int32).at[dest].set(pair_tok)
    blk_start = jnp.arange(NB, dtype=jnp.int32) * BM
    block_expert = (jnp.sum(
        (blk_start[:, None] >= poff[None, :]).astype(jnp.int32), axis=1) - 1
    ).astype(jnp.int32)

    xs = jnp.take(xb, src_row, axis=0)

    eo = _gmm_chunk(xs, ew1, eb1.reshape(E, 1, H),
                    ew2, eb2.reshape(E, 1, H), block_expert, NB, 0, None)

    d2 = dest.reshape(n, K)
    comb_idx = jnp.concatenate([d2[:, 0], d2[:, 1]])
    comb = jnp.take(eo, comb_idx, axis=0)

    out = _combine_mlp(comb, g01, mw1, mb1.reshape(1, H),
                       mw2, mb2.reshape(1, H), fw,
                       fb.reshape(1, D_OUT), n, 0, None)

    aux = LOSS_COEF * (_cv_squared(imp.reshape(E)) +
                       _cv_squared(load.reshape(E)))
    return (out, aux)


# speedup vs baseline: 1.2062x; 1.0146x over previous
import functools

import jax
import jax.numpy as jnp
from jax import lax
from jax.experimental import pallas as pl
from jax.experimental.pallas import tpu as pltpu

N_TOK = 4096
D_IN = 1024
E = 8
H = 1024
D_OUT = 1024
K = 2
LOSS_COEF = 0.01

BN_GATE = 1024
BM = 256
NB = 39
P = NB * BM
PB1 = 20
BN_MLP = 1024

_BF = jnp.bfloat16
_F32 = jnp.float32


def _gating_body(x_ref, wg_ref, i01_ref, g01_ref, rank01_ref, imp_ref,
                 load_ref, cnt_ref, xb_ref):
    t = pl.program_id(0)
    xbf = x_ref[...].astype(_BF)
    xb_ref[...] = xbf
    logits = jnp.dot(xbf, wg_ref[...].astype(_BF),
                     preferred_element_type=_F32)
    bn = logits.shape[0]
    iot = lax.broadcasted_iota(jnp.int32, (bn, E), 1)
    v0 = jnp.max(logits, axis=-1, keepdims=True)
    i0 = jnp.argmax(logits, axis=-1).astype(jnp.int32)
    masked = jnp.where(iot == i0[:, None], -jnp.inf, logits)
    v1 = jnp.max(masked, axis=-1, keepdims=True)
    i1 = jnp.argmax(masked, axis=-1).astype(jnp.int32)
    e1 = jnp.exp(v1 - v0)
    denom = 1.0 + e1
    g0 = 1.0 / denom
    g1 = e1 / denom
    i01_ref[...] = jnp.concatenate([i0[:, None], i1[:, None]], axis=1)
    g01_ref[...] = jnp.concatenate([g0, g1], axis=1)
    oh0 = (iot == i0[:, None]).astype(_F32)
    oh1 = (iot == i1[:, None]).astype(_F32)
    imp_blk = jnp.sum(oh0 * g0 + oh1 * g1, axis=0, keepdims=True)
    load_blk = jnp.sum(oh0 + oh1 * (g1 > 0.0).astype(_F32), axis=0,
                       keepdims=True)

    @pl.when(t == 0)
    def _():
        imp_ref[...] = jnp.zeros_like(imp_ref)
        load_ref[...] = jnp.zeros_like(load_ref)
        cnt_ref[...] = jnp.zeros_like(cnt_ref)

    base = cnt_ref[...]
    ohb = oh0 + oh1
    row = lax.broadcasted_iota(jnp.int32, (bn, bn), 0)
    col = lax.broadcasted_iota(jnp.int32, (bn, bn), 1)
    ltri = (row > col).astype(_F32)
    cex = jnp.dot(ltri, ohb, preferred_element_type=_F32)
    rank0 = jnp.sum((base + cex) * oh0, axis=1, keepdims=True)
    rank1 = jnp.sum((base + cex) * oh1, axis=1, keepdims=True)
    rank01_ref[...] = jnp.concatenate([rank0, rank1],
                                      axis=1).astype(jnp.int32)

    imp_ref[...] += imp_blk
    load_ref[...] += load_blk
    cnt_ref[...] += jnp.sum(ohb, axis=0, keepdims=True)


def _gating(x, w_gate):
    n = x.shape[0]
    grid = (n // BN_GATE,)
    return pl.pallas_call(
        _gating_body,
        grid=grid,
        in_specs=[
            pl.BlockSpec((BN_GATE, D_IN), lambda t: (t, 0)),
            pl.BlockSpec((D_IN, E), lambda t: (0, 0)),
        ],
        out_specs=[
            pl.BlockSpec((BN_GATE, K), lambda t: (t, 0)),
            pl.BlockSpec((BN_GATE, K), lambda t: (t, 0)),
            pl.BlockSpec((BN_GATE, K), lambda t: (t, 0)),
            pl.BlockSpec((1, E), lambda t: (0, 0)),
            pl.BlockSpec((1, E), lambda t: (0, 0)),
            pl.BlockSpec((1, E), lambda t: (0, 0)),
            pl.BlockSpec((BN_GATE, D_IN), lambda t: (t, 0)),
        ],
        out_shape=[
            jax.ShapeDtypeStruct((n, K), jnp.int32),
            jax.ShapeDtypeStruct((n, K), _F32),
            jax.ShapeDtypeStruct((n, K), jnp.int32),
            jax.ShapeDtypeStruct((1, E), _F32),
            jax.ShapeDtypeStruct((1, E), _F32),
            jax.ShapeDtypeStruct((1, E), _F32),
            jax.ShapeDtypeStruct((n, D_IN), _BF),
        ],
        compiler_params=pltpu.CompilerParams(
            dimension_semantics=("arbitrary",)),
    )(x, w_gate)


def _gmm_body(be_ref, xs_ref, w1_ref, b1_ref, w2_ref, b2_ref, eo_ref,
              w1s_ref, w2s_ref):
    i = pl.program_id(0)
    prev = be_ref[jnp.maximum(i - 1, 0)]

    @pl.when((i == 0) | (be_ref[i] != prev))
    def _():
        w1s_ref[...] = w1_ref[0].astype(_BF)
        w2s_ref[...] = w2_ref[0].astype(_BF)

    h = jnp.dot(xs_ref[...], w1s_ref[...], preferred_element_type=_F32)
    h = jnp.tanh(h + b1_ref[0])
    eo = jnp.dot(h.astype(_BF), w2s_ref[...], preferred_element_type=_F32)
    eo_ref[...] = (eo + b2_ref[0]).astype(_BF)


def _gmm_chunk(xs, ew1, eb1, ew2, eb2, block_expert, nb, base, eo_prev):
    grid_spec = pltpu.PrefetchScalarGridSpec(
        num_scalar_prefetch=1,
        grid=(nb,),
        in_specs=[
            pl.BlockSpec((BM, D_IN), lambda i, be: (i, 0)),
            pl.BlockSpec((1, D_IN, H), lambda i, be: (be[i], 0, 0)),
            pl.BlockSpec((1, 1, H), lambda i, be: (be[i], 0, 0)),
            pl.BlockSpec((1, H, H), lambda i, be: (be[i], 0, 0)),
            pl.BlockSpec((1, 1, H), lambda i, be: (be[i], 0, 0)),
        ] + ([pl.BlockSpec(memory_space=pl.ANY)]
             if eo_prev is not None else []),
        out_specs=pl.BlockSpec((BM, H), lambda i, be: (base + i, 0)),
        scratch_shapes=[pltpu.VMEM((D_IN, H), _BF), pltpu.VMEM((H, H), _BF)],
    )
    args = [block_expert, xs, ew1, eb1, ew2, eb2]
    kwargs = {}
    if eo_prev is not None:
        args.append(eo_prev)
        kwargs["input_output_aliases"] = {6: 0}

    def body(be_ref, xs_ref, w1_ref, b1_ref, w2_ref, b2_ref, *rest):
        w1s_ref, w2s_ref = rest[-2], rest[-1]
        eo_ref = rest[-3]
        _gmm_body(be_ref, xs_ref, w1_ref, b1_ref, w2_ref, b2_ref, eo_ref,
                  w1s_ref, w2s_ref)

    return pl.pallas_call(
        body,
        grid_spec=grid_spec,
        out_shape=jax.ShapeDtypeStruct((P, H), _BF),
        compiler_params=pltpu.CompilerParams(
            dimension_semantics=("arbitrary",)),
        **kwargs,
    )(*args)


def _mlp_body(a_ref, b_ref, g_ref, mw1_ref, mb1_ref, mw2_ref, mb2_ref,
              fw_ref, fb_ref, out_ref, w1s_ref, w2s_ref, fws_ref):
    @pl.when(pl.program_id(0) == 0)
    def _():
        w1s_ref[...] = mw1_ref[...].astype(_BF)
        w2s_ref[...] = mw2_ref[...].astype(_BF)
        fws_ref[...] = fw_ref[...].astype(_BF)

    g = g_ref[...].astype(_BF).astype(_F32)
    g0 = g[:, 0:1]
    g1 = g[:, 1:2]
    y = a_ref[...].astype(_F32) * g0 + b_ref[...].astype(_F32) * g1
    y1 = jnp.tanh(jnp.dot(y.astype(_BF), w1s_ref[...],
                          preferred_element_type=_F32) + mb1_ref[...])
    y2 = jnp.tanh(jnp.dot(y1.astype(_BF), w2s_ref[...],
                          preferred_element_type=_F32) + mb2_ref[...])
    out_ref[...] = jnp.dot(y2.astype(_BF), fws_ref[...],
                           preferred_element_type=_F32) + fb_ref[...]


def _combine_mlp(comb, g01, mw1, mb1, mw2, mb2, fw, fb, n_total, base,
                 out_prev):
    nc = comb.shape[0] // 2
    nblk = nc // BN_MLP
    grid = (nblk,)

    def body(a_ref, b_ref, g_ref, mw1_ref, mb1_ref, mw2_ref, mb2_ref,
             fw_ref, fb_ref, *rest):
        _mlp_body(a_ref, b_ref, g_ref, mw1_ref, mb1_ref, mw2_ref, mb2_ref,
                  fw_ref, fb_ref, rest[-4], rest[-3], rest[-2], rest[-1])

    in_specs = [
        pl.BlockSpec((BN_MLP, H), lambda t: (t, 0)),
        pl.BlockSpec((BN_MLP, H), lambda t: (nblk + t, 0)),
        pl.BlockSpec((BN_MLP, K), lambda t: (t, 0)),
        pl.BlockSpec((H, H), lambda t: (0, 0)),
        pl.BlockSpec((1, H), lambda t: (0, 0)),
        pl.BlockSpec((H, H), lambda t: (0, 0)),
        pl.BlockSpec((1, H), lambda t: (0, 0)),
        pl.BlockSpec((H, D_OUT), lambda t: (0, 0)),
        pl.BlockSpec((1, D_OUT), lambda t: (0, 0)),
    ]
    args = [comb, comb, g01, mw1, mb1, mw2, mb2, fw, fb]
    kwargs = {}
    if out_prev is not None:
        in_specs.append(pl.BlockSpec(memory_space=pl.ANY))
        args.append(out_prev)
        kwargs["input_output_aliases"] = {9: 0}
    return pl.pallas_call(
        body,
        grid=grid,
        in_specs=in_specs,
        out_specs=pl.BlockSpec((BN_MLP, D_OUT), lambda t: (base + t, 0)),
        out_shape=jax.ShapeDtypeStruct((n_total, D_OUT), _F32),
        scratch_shapes=[pltpu.VMEM((H, H), _BF), pltpu.VMEM((H, H), _BF),
                        pltpu.VMEM((H, D_OUT), _BF)],
        compiler_params=pltpu.CompilerParams(
            dimension_semantics=("arbitrary",)),
        **kwargs,
    )(*args)


def _cv_squared(v):
    eps = 1e-10
    return jnp.var(v, ddof=1) / (jnp.mean(v) ** 2 + eps)


def kernel(x, w_gate, ew1, eb1, ew2, eb2, mw1, mb1, mw2, mb2, fw, fb):
    n = x.shape[0]
    i01, g01, rank01, imp, load, cnt, xb = _gating(x, w_gate)

    flat_e = i01.reshape(-1)
    counts = cnt.reshape(E).astype(jnp.int32)
    pc = ((counts + BM - 1) // BM) * BM
    poff = jnp.concatenate([jnp.zeros((1,), jnp.int32),
                            jnp.cumsum(pc)[:-1].astype(jnp.int32)])
    eoh = (flat_e[:, None] == jnp.arange(E, dtype=jnp.int32)[None, :])
    dest = (jnp.sum(eoh * poff[None, :], axis=1, dtype=jnp.int32)
            + rank01.reshape(-1))
    pair_tok = jnp.arange(n * K, dtype=jnp.int32) // K
    src_row = jnp.zeros((P,), jnp.int32).at[dest].set(pair_tok)
    blk_start = jnp.arange(NB, dtype=jnp.int32) * BM
    block_expert = (jnp.sum(
        (blk_start[:, None] >= poff[None, :]).astype(jnp.int32), axis=1) - 1
    ).astype(jnp.int32)

    xs = jnp.take(xb, src_row, axis=0)

    eo = _gmm_chunk(xs, ew1, eb1.reshape(E, 1, H),
                    ew2, eb2.reshape(E, 1, H), block_expert, NB, 0, None)

    d2 = dest.reshape(n, K)
    comb_idx = jnp.concatenate([d2[:, 0], d2[:, 1]])
    comb = jnp.take(eo, comb_idx, axis=0)

    out = _combine_mlp(comb, g01, mw1, mb1.reshape(1, H),
                       mw2, mb2.reshape(1, H), fw,
                       fb.reshape(1, D_OUT), n, 0, None)

    aux = LOSS_COEF * (_cv_squared(imp.reshape(E)) +
                       _cv_squared(load.reshape(E)))
    return (out, aux)


# speedup vs baseline: 1.2069x; 1.0006x over previous
import jax
import jax.numpy as jnp
from jax import lax
from jax.experimental import pallas as pl
from jax.experimental.pallas import tpu as pltpu

N_TOK = 4096
D_IN = 1024
E = 8
H = 1024
D_OUT = 1024
K = 2
LOSS_COEF = 0.01

BN_GATE = 1024
BM = 256
NB = 39
P = NB * BM
BN_MLP = 1024

_BF = jnp.bfloat16
_F32 = jnp.float32


def _gating_body(x_ref, wg_ref, i01_ref, g01_ref, rank01_ref, imp_ref,
                 load_ref, cnt_ref, xb_ref):
    t = pl.program_id(0)
    xbf = x_ref[...].astype(_BF)
    xb_ref[...] = xbf
    logits = jnp.dot(xbf, wg_ref[...].astype(_BF),
                     preferred_element_type=_F32)
    bn = logits.shape[0]
    iot = lax.broadcasted_iota(jnp.int32, (bn, E), 1)
    v0 = jnp.max(logits, axis=-1, keepdims=True)
    i0 = jnp.argmax(logits, axis=-1).astype(jnp.int32)
    masked = jnp.where(iot == i0[:, None], -jnp.inf, logits)
    v1 = jnp.max(masked, axis=-1, keepdims=True)
    i1 = jnp.argmax(masked, axis=-1).astype(jnp.int32)
    e1 = jnp.exp(v1 - v0)
    denom = 1.0 + e1
    g0 = 1.0 / denom
    g1 = e1 / denom
    i01_ref[...] = jnp.concatenate([i0[:, None], i1[:, None]], axis=1)
    g01_ref[...] = jnp.concatenate([g0, g1], axis=1)
    oh0 = (iot == i0[:, None]).astype(_F32)
    oh1 = (iot == i1[:, None]).astype(_F32)
    imp_blk = jnp.sum(oh0 * g0 + oh1 * g1, axis=0, keepdims=True)
    load_blk = jnp.sum(oh0 + oh1 * (g1 > 0.0).astype(_F32), axis=0,
                       keepdims=True)

    @pl.when(t == 0)
    def _():
        imp_ref[...] = jnp.zeros_like(imp_ref)
        load_ref[...] = jnp.zeros_like(load_ref)
        cnt_ref[...] = jnp.zeros_like(cnt_ref)

    base = cnt_ref[...]
    ohb = oh0 + oh1
    row = lax.broadcasted_iota(jnp.int32, (bn, bn), 0)
    col = lax.broadcasted_iota(jnp.int32, (bn, bn), 1)
    ltri = (row > col).astype(_F32)
    cex = jnp.dot(ltri, ohb, preferred_element_type=_F32)
    rank0 = jnp.sum((base + cex) * oh0, axis=1, keepdims=True)
    rank1 = jnp.sum((base + cex) * oh1, axis=1, keepdims=True)
    rank01_ref[...] = jnp.concatenate([rank0, rank1],
                                      axis=1).astype(jnp.int32)

    imp_ref[...] += imp_blk
    load_ref[...] += load_blk
    cnt_ref[...] += jnp.sum(ohb, axis=0, keepdims=True)


def _gating(x, w_gate):
    n = x.shape[0]
    grid = (n // BN_GATE,)
    return pl.pallas_call(
        _gating_body,
        grid=grid,
        in_specs=[
            pl.BlockSpec((BN_GATE, D_IN), lambda t: (t, 0)),
            pl.BlockSpec((D_IN, E), lambda t: (0, 0)),
        ],
        out_specs=[
            pl.BlockSpec((BN_GATE, K), lambda t: (t, 0)),
            pl.BlockSpec((BN_GATE, K), lambda t: (t, 0)),
            pl.BlockSpec((BN_GATE, K), lambda t: (t, 0)),
            pl.BlockSpec((1, E), lambda t: (0, 0)),
            pl.BlockSpec((1, E), lambda t: (0, 0)),
            pl.BlockSpec((1, E), lambda t: (0, 0)),
            pl.BlockSpec((BN_GATE, D_IN), lambda t: (t, 0)),
        ],
        out_shape=[
            jax.ShapeDtypeStruct((n, K), jnp.int32),
            jax.ShapeDtypeStruct((n, K), _F32),
            jax.ShapeDtypeStruct((n, K), jnp.int32),
            jax.ShapeDtypeStruct((1, E), _F32),
            jax.ShapeDtypeStruct((1, E), _F32),
            jax.ShapeDtypeStruct((1, E), _F32),
            jax.ShapeDtypeStruct((n, D_IN), _BF),
        ],
        compiler_params=pltpu.CompilerParams(
            dimension_semantics=("arbitrary",)),
    )(x, w_gate)


def _gmm_body(be_ref, xs_ref, w1_ref, b1_ref, w2_ref, b2_ref, eo_ref,
              w1s_ref, w2s_ref):
    i = pl.program_id(0)
    prev = be_ref[jnp.maximum(i - 1, 0)]

    @pl.when((i == 0) | (be_ref[i] != prev))
    def _():
        w1s_ref[...] = w1_ref[0].astype(_BF)
        w2s_ref[...] = w2_ref[0].astype(_BF)

    h = jnp.dot(xs_ref[...], w1s_ref[...], preferred_element_type=_F32)
    h = jnp.tanh(h + b1_ref[0])
    eo = jnp.dot(h.astype(_BF), w2s_ref[...], preferred_element_type=_F32)
    eo_ref[...] = (eo + b2_ref[0]).astype(_BF)


def _gmm(xs, ew1, eb1, ew2, eb2, block_expert):
    grid_spec = pltpu.PrefetchScalarGridSpec(
        num_scalar_prefetch=1,
        grid=(NB,),
        in_specs=[
            pl.BlockSpec((BM, D_IN), lambda i, be: (i, 0)),
            pl.BlockSpec((1, D_IN, H), lambda i, be: (be[i], 0, 0)),
            pl.BlockSpec((1, 1, H), lambda i, be: (be[i], 0, 0)),
            pl.BlockSpec((1, H, H), lambda i, be: (be[i], 0, 0)),
            pl.BlockSpec((1, 1, H), lambda i, be: (be[i], 0, 0)),
        ],
        out_specs=pl.BlockSpec((BM, H), lambda i, be: (i, 0)),
        scratch_shapes=[pltpu.VMEM((D_IN, H), _BF), pltpu.VMEM((H, H), _BF)],
    )
    return pl.pallas_call(
        _gmm_body,
        grid_spec=grid_spec,
        out_shape=jax.ShapeDtypeStruct((P, H), _BF),
        compiler_params=pltpu.CompilerParams(
            dimension_semantics=("arbitrary",)),
    )(block_expert, xs, ew1, eb1, ew2, eb2)


def _mlp_body(a_ref, b_ref, g_ref, mw1_ref, mb1_ref, mw2_ref, mb2_ref,
              fw_ref, fb_ref, out_ref, w1s_ref, w2s_ref, fws_ref):
    @pl.when(pl.program_id(0) == 0)
    def _():
        w1s_ref[...] = mw1_ref[...].astype(_BF)
        w2s_ref[...] = mw2_ref[...].astype(_BF)
        fws_ref[...] = fw_ref[...].astype(_BF)

    g = g_ref[...].astype(_BF).astype(_F32)
    g0 = g[:, 0:1]
    g1 = g[:, 1:2]
    y = a_ref[...].astype(_F32) * g0 + b_ref[...].astype(_F32) * g1
    y1 = jnp.tanh(jnp.dot(y.astype(_BF), w1s_ref[...],
                          preferred_element_type=_F32) + mb1_ref[...])
    y2 = jnp.tanh(jnp.dot(y1.astype(_BF), w2s_ref[...],
                          preferred_element_type=_F32) + mb2_ref[...])
    out_ref[...] = jnp.dot(y2.astype(_BF), fws_ref[...],
                           preferred_element_type=_F32) + fb_ref[...]


def _combine_mlp(comb, g01, mw1, mb1, mw2, mb2, fw, fb):
    nc = comb.shape[0] // 2
    nblk = nc // BN_MLP
    in_specs = [
        pl.BlockSpec((BN_MLP, H), lambda t: (t, 0)),
        pl.BlockSpec((BN_MLP, H), lambda t: (nblk + t, 0)),
        pl.BlockSpec((BN_MLP, K), lambda t: (t, 0)),
        pl.BlockSpec((H, H), lambda t: (0, 0)),
        pl.BlockSpec((1, H), lambda t: (0, 0)),
        pl.BlockSpec((H, H), lambda t: (0, 0)),
        pl.BlockSpec((1, H), lambda t: (0, 0)),
        pl.BlockSpec((H, D_OUT), lambda t: (0, 0)),
        pl.BlockSpec((1, D_OUT), lambda t: (0, 0)),
    ]
    return pl.pallas_call(
        _mlp_body,
        grid=(nblk,),
        in_specs=in_specs,
        out_specs=pl.BlockSpec((BN_MLP, D_OUT), lambda t: (t, 0)),
        out_shape=jax.ShapeDtypeStruct((nc, D_OUT), _F32),
        scratch_shapes=[pltpu.VMEM((H, H), _BF), pltpu.VMEM((H, H), _BF),
                        pltpu.VMEM((H, D_OUT), _BF)],
        compiler_params=pltpu.CompilerParams(
            dimension_semantics=("arbitrary",)),
    )(comb, comb, g01, mw1, mb1, mw2, mb2, fw, fb)


def _cv_squared(v):
    eps = 1e-10
    return jnp.var(v, ddof=1) / (jnp.mean(v) ** 2 + eps)


def kernel(x, w_gate, ew1, eb1, ew2, eb2, mw1, mb1, mw2, mb2, fw, fb):
    n = x.shape[0]
    i01, g01, rank01, imp, load, cnt, xb = _gating(x, w_gate)

    flat_e = i01.reshape(-1)
    counts = cnt.reshape(E).astype(jnp.int32)
    pc = ((counts + BM - 1) // BM) * BM
    poff = jnp.concatenate([jnp.zeros((1,), jnp.int32),
                            jnp.cumsum(pc)[:-1].astype(jnp.int32)])
    eoh = (flat_e[:, None] == jnp.arange(E, dtype=jnp.int32)[None, :])
    dest = (jnp.sum(eoh * poff[None, :], axis=1, dtype=jnp.int32)
            + rank01.reshape(-1))
    pair_tok = jnp.arange(n * K, dtype=jnp.int32) // K
    src_row = jnp.zeros((P,), jnp.int32).at[dest].set(pair_tok)
    blk_start = jnp.arange(NB, dtype=jnp.int32) * BM
    block_expert = (jnp.sum(
        (blk_start[:, None] >= poff[None, :]).astype(jnp.int32), axis=1) - 1
    ).astype(jnp.int32)

    xs = jnp.take(xb, src_row, axis=0)

    eo = _gmm(xs, ew1, eb1.reshape(E, 1, H),
              ew2, eb2.reshape(E, 1, H), block_expert)

    d2 = dest.reshape(n, K)
    comb_idx = jnp.concatenate([d2[:, 0], d2[:, 1]])
    comb = jnp.take(eo, comb_idx, axis=0)

    out = _combine_mlp(comb, g01, mw1, mb1.reshape(1, H),
                       mw2, mb2.reshape(1, H), fw, fb.reshape(1, D_OUT))

    aux = LOSS_COEF * (_cv_squared(imp.reshape(E)) +
                       _cv_squared(load.reshape(E)))
    return (out, aux)
